# Initial kernel scaffold; baseline (speedup 1.0000x reference)
import jax
import jax.numpy as jnp
from jax.experimental import pallas as pl


def kernel(query_points, source_vertices, source_normals):
    raise NotImplementedError("write your pallas kernel here")


# speedup vs baseline: 14.7353x; 14.7353x over previous
import jax
import jax.numpy as jnp
from jax.experimental import pallas as pl

_K = 256
_SIGMA_N = 0.8
_EPS = 1e-8
_QT = 128
_SCW = 2048
_NBS = 22


def _rimls_kernel(q_ref, pT_ref, nT_ref, f_ref, g_ref):
    qt = q_ref.shape[0]
    ns = pT_ref.shape[1]
    q = q_ref[...]
    pT = pT_ref[...]
    nT_raw = nT_ref[...]

    nn = jnp.sqrt(jnp.sum(nT_raw * nT_raw, axis=0, keepdims=True))
    nT = nT_raw / jnp.maximum(nn, _EPS)

    qx = q[:, 0:1]
    qy = q[:, 1:2]
    qz = q[:, 2:3]
    q2 = qx * qx + qy * qy + qz * qz
    s2 = jnp.sum(pT * pT, axis=0, keepdims=True)
    def _b(v):
        return v.astype(jnp.bfloat16).astype(jnp.float32)
    qp = (_b(qx) * _b(pT[0:1, :]) + _b(qy) * _b(pT[1:2, :])
          + _b(qz) * _b(pT[2:3, :]))
    d2 = q2 + s2 - 2.0 * qp
    d2 = jnp.maximum(d2, 0.0)

    hi0 = jnp.max(d2, axis=1, keepdims=True)
    lo0 = jnp.zeros_like(hi0)
    kf = jnp.float32(_K)

    def bs_body(_, c):
        lo, hi = c
        mid = 0.5 * (lo + hi)
        cnt = jnp.sum((d2 <= mid).astype(jnp.float32), axis=1, keepdims=True)
        ge = cnt >= kf
        return jnp.where(ge, lo, mid), jnp.where(ge, mid, hi)

    _, T = jax.lax.fori_loop(0, _NBS, bs_body, (lo0, hi0))
    sd = jnp.sqrt(d2)
    below = d2 < T
    cnt_lt = jnp.sum(below.astype(jnp.float32), axis=1, keepdims=True)
    sum_lt = jnp.sum(jnp.where(below, sd, 0.0), axis=1, keepdims=True)
    ksum = sum_lt + (kf - cnt_lt) * jnp.sqrt(T)
    h = ksum * (1.0 / _K) + _EPS

    rh2 = 1.0 / (h * h)
    isr = 1.0 / (0.5 * h + _EPS)
    isn2 = 1.0 / (_SIGMA_N * _SIGMA_N)
    nch = ns // _SCW

    def fit_pass(prev):
        z = jnp.zeros((qt, 1), jnp.float32)
        acc = [z] * 11
        for c in range(nch):
            s = c * _SCW
            p_x = pT[0:1, s:s + _SCW]
            p_y = pT[1:2, s:s + _SCW]
            p_z = pT[2:3, s:s + _SCW]
            n_x = nT[0:1, s:s + _SCW]
            n_y = nT[1:2, s:s + _SCW]
            n_z = nT[2:3, s:s + _SCW]
            px = qx - p_x
            py = qy - p_y
            pz = qz - p_z
            fx = px * n_x + py * n_y + pz * n_z
            r2 = px * px + py * py + pz * pz
            t = jnp.maximum(1.0 - r2 * rh2, 0.0)
            t2 = t * t
            phi = t2 * t2
            m = (-8.0 * rh2) * (t2 * t)
            if prev is None:
                w = phi
                cc = m
            else:
                f_p, gx_p, gy_p, gz_p = prev
                u = (fx - f_p) * isr
                dnx = n_x - gx_p
                dny = n_y - gy_p
                dnz = n_z - gz_p
                a = jnp.exp(-(u * u)
                            - (dnx * dnx + dny * dny + dnz * dnz) * isn2)
                w = a * phi
                cc = a * m
            ex = cc * px
            ey = cc * py
            ez = cc * pz
            terms = (w, w * fx, ex, ey, ez, ex * fx, ey * fx, ez * fx,
                     w * n_x, w * n_y, w * n_z)
            acc = [a0 + jnp.sum(tm, axis=1, keepdims=True)
                   for a0, tm in zip(acc, terms)]
        sw, wfx, sex, sey, sez, sexf, seyf, sezf, wnx, wny, wnz = acc
        sumW = sw + _EPS
        f_new = wfx / sumW
        gx = (sexf - f_new * sex + wnx) / sumW
        gy = (seyf - f_new * sey + wny) / sumW
        gz = (sezf - f_new * sez + wnz) / sumW
        return f_new, gx, gy, gz

    out0 = fit_pass(None)
    f1, gx1, gy1, gz1 = fit_pass(out0)
    f_ref[...] = f1
    g_ref[:, 0:1] = gx1
    g_ref[:, 1:2] = gy1
    g_ref[:, 2:3] = gz1


def kernel(query_points, source_vertices, source_normals):
    nq = query_points.shape[0]
    ns = source_vertices.shape[0]
    pT = source_vertices.T
    nT = source_normals.T
    f2, g = pl.pallas_call(
        _rimls_kernel,
        grid=(nq // _QT,),
        in_specs=[
            pl.BlockSpec((_QT, 3), lambda i: (i, 0)),
            pl.BlockSpec((3, ns), lambda i: (0, 0)),
            pl.BlockSpec((3, ns), lambda i: (0, 0)),
        ],
        out_specs=[
            pl.BlockSpec((_QT, 1), lambda i: (i, 0)),
            pl.BlockSpec((_QT, 3), lambda i: (i, 0)),
        ],
        out_shape=[
            jax.ShapeDtypeStruct((nq, 1), jnp.float32),
            jax.ShapeDtypeStruct((nq, 3), jnp.float32),
        ],
    )(query_points, pT, nT)
    return f2[:, 0], g
